# baseline (speedup 1.0000x reference)
import jax
import jax.numpy as jnp
from jax import lax
from jax.experimental import pallas as pl
from jax.experimental.pallas import tpu as pltpu, tpu_sc as plsc

B = 1048576
N_T = 8
N_S = 8
TBL = N_T * N_S * N_T * N_S

NC = 2
NS = 16
L = 16
NW = NC * NS
PER_W = B // NW
CHUNK = 4096
N_CHUNKS = PER_W // CHUNK
N_PAIRS = N_CHUNKS // 2


def _make_sc_call():
    mesh = plsc.VectorSubcoreMesh(core_axis_name="c", subcore_axis_name="s")

    chunk_i32 = pltpu.VMEM((CHUNK,), jnp.int32)

    def sc_kernel(tp_hbm, sp_hbm, t_hbm, s_hbm, obs_hbm, alpha_hbm, beta_hbm,
                  out_hbm,
                  table_v, obs_v, alpha_v, beta_v, decay_v,
                  tp0, sp0, t0, s0,
                  tp1, sp1, t1, s1,
                  out_v,
                  sem_p, sem_a, sem_b, sem_out):
        wid = lax.axis_index("s") * NC + lax.axis_index("c")
        banks = ((tp0, sp0, t0, s0), (tp1, sp1, t1, s1))
        srcs = (tp_hbm, sp_hbm, t_hbm, s_hbm)

        def in_copies(c, bank, sem):
            base = wid * PER_W + c * CHUNK
            return [pltpu.make_async_copy(src.at[pl.ds(base, CHUNK)], dst, sem)
                    for src, dst in zip(srcs, bank)]

        c1 = pltpu.make_async_copy(obs_hbm, obs_v, sem_p)
        c2 = pltpu.make_async_copy(alpha_hbm, alpha_v, sem_p)
        c3 = pltpu.make_async_copy(beta_hbm, beta_v, sem_p)
        c1.start(); c2.start(); c3.start()
        for cp in in_copies(0, banks[0], sem_a):
            cp.start()
        prime = pltpu.make_async_copy(out_hbm.at[pl.ds(wid * PER_W, CHUNK)],
                                      out_v, sem_out)
        prime.start()
        c1.wait(); c2.wait(); c3.wait()

        lane = lax.iota(jnp.int32, L)
        beta = plsc.load_gather(beta_v, [lane & 0])
        decay_v[pl.ds(0, L)] = beta * jnp.exp(-beta * lane.astype(jnp.float32))

        zeros = jnp.zeros((L,), jnp.float32)

        @plsc.parallel_loop(0, TBL // L, unroll=8)
        def _zero(i):
            table_v[pl.ds(i * L, L)] = zeros

        @plsc.parallel_loop(0, N_T * N_S * N_T // L, unroll=4)
        def _build(i):
            e = i * L + lane
            tp_i = e >> 6
            sp_i = (e >> 3) & 7
            t_i = e & 7
            obs_g = plsc.load_gather(obs_v, [tp_i, sp_i]).astype(jnp.float32)
            al_g = plsc.load_gather(alpha_v, [sp_i, sp_i])
            dk = plsc.load_gather(decay_v, [jnp.abs(t_i - tp_i)])
            plsc.store_scatter(table_v, [e * N_S + sp_i], al_g * obs_g * dk)

        def do_chunk(c, bank, sem):
            for cp in in_copies(c, bank, sem):
                cp.wait()
            pltpu.make_async_copy(out_hbm.at[pl.ds(wid * PER_W, CHUNK)],
                                  out_v, sem_out).wait()
            tp_v, sp_v, t_v, s_v = bank

            @plsc.parallel_loop(0, CHUNK // L, unroll=8)
            def _gather(k):
                sl = pl.ds(k * L, L)
                flat = ((tp_v[sl] * N_S + sp_v[sl]) * N_T + t_v[sl]) * N_S + s_v[sl]
                out_v[sl] = plsc.load_gather(table_v, [flat])

            base = wid * PER_W + c * CHUNK
            pltpu.make_async_copy(out_v, out_hbm.at[pl.ds(base, CHUNK)],
                                  sem_out).start()

        def pair(j, _):
            c0 = 2 * j
            for cp in in_copies(c0 + 1, banks[1], sem_b):
                cp.start()
            do_chunk(c0, banks[0], sem_a)

            @pl.when(c0 + 2 < N_CHUNKS)
            def _():
                for cp in in_copies(c0 + 2, banks[0], sem_a):
                    cp.start()
            do_chunk(c0 + 1, banks[1], sem_b)
            return 0

        lax.fori_loop(0, N_PAIRS, pair, 0)
        pltpu.make_async_copy(
            out_v, out_hbm.at[pl.ds(wid * PER_W + (N_CHUNKS - 1) * CHUNK, CHUNK)],
            sem_out).wait()

    return pl.kernel(
        sc_kernel,
        out_type=jax.ShapeDtypeStruct((B,), jnp.float32),
        mesh=mesh,
        compiler_params=pltpu.CompilerParams(needs_layout_passes=False),
        scratch_types=[
            pltpu.VMEM((TBL,), jnp.float32),
            pltpu.VMEM((N_T, N_S), jnp.int32),
            pltpu.VMEM((N_S, N_S), jnp.float32),
            pltpu.VMEM((1,), jnp.float32),
            pltpu.VMEM((L,), jnp.float32),
            chunk_i32, chunk_i32, chunk_i32, chunk_i32,
            chunk_i32, chunk_i32, chunk_i32, chunk_i32,
            pltpu.VMEM((CHUNK,), jnp.float32),
            pltpu.SemaphoreType.DMA,
            pltpu.SemaphoreType.DMA,
            pltpu.SemaphoreType.DMA,
            pltpu.SemaphoreType.DMA,
        ],
    )


_SC_CALL = _make_sc_call()


def kernel(tp, sp, t, s, obs, alpha, beta):
    return _SC_CALL(tp, sp, t, s, obs, alpha, beta)


# speedup vs baseline: 1.0134x; 1.0134x over previous
import jax
import jax.numpy as jnp
from jax import lax
from jax.experimental import pallas as pl
from jax.experimental.pallas import tpu as pltpu, tpu_sc as plsc

B = 1048576
N_T = 8
N_S = 8
TBL = N_T * N_S * N_T * N_S

NC = 2
NS = 16
L = 16
NW = NC * NS
PER_W = B // NW
CHUNK = 8192
N_CHUNKS = PER_W // CHUNK
N_PAIRS = N_CHUNKS // 2


def _make_sc_call():
    mesh = plsc.VectorSubcoreMesh(core_axis_name="c", subcore_axis_name="s")

    chunk_i32 = pltpu.VMEM((CHUNK,), jnp.int32)

    def sc_kernel(tp_hbm, sp_hbm, t_hbm, s_hbm, obs_hbm, alpha_hbm, beta_hbm,
                  out_hbm,
                  table_v, obs_v, alpha_v, beta_v, decay_v,
                  tp0, sp0, t0, s0,
                  tp1, sp1, t1, s1,
                  out_v,
                  sem_p, sem_a, sem_b, sem_out):
        wid = lax.axis_index("s") * NC + lax.axis_index("c")
        banks = ((tp0, sp0, t0, s0), (tp1, sp1, t1, s1))
        srcs = (tp_hbm, sp_hbm, t_hbm, s_hbm)

        def in_copies(c, bank, sem):
            base = wid * PER_W + c * CHUNK
            return [pltpu.make_async_copy(src.at[pl.ds(base, CHUNK)], dst, sem)
                    for src, dst in zip(srcs, bank)]

        c1 = pltpu.make_async_copy(obs_hbm, obs_v, sem_p)
        c2 = pltpu.make_async_copy(alpha_hbm, alpha_v, sem_p)
        c3 = pltpu.make_async_copy(beta_hbm, beta_v, sem_p)
        c1.start(); c2.start(); c3.start()
        for cp in in_copies(0, banks[0], sem_a):
            cp.start()
        prime = pltpu.make_async_copy(out_hbm.at[pl.ds(wid * PER_W, CHUNK)],
                                      out_v, sem_out)
        prime.start()
        c1.wait(); c2.wait(); c3.wait()

        lane = lax.iota(jnp.int32, L)
        beta = plsc.load_gather(beta_v, [lane & 0])
        decay_v[pl.ds(0, L)] = beta * jnp.exp(-beta * lane.astype(jnp.float32))

        zeros = jnp.zeros((L,), jnp.float32)

        @plsc.parallel_loop(0, TBL // L, unroll=8)
        def _zero(i):
            table_v[pl.ds(i * L, L)] = zeros

        @plsc.parallel_loop(0, N_T * N_S * N_T // L, unroll=4)
        def _build(i):
            e = i * L + lane
            tp_i = e >> 6
            sp_i = (e >> 3) & 7
            t_i = e & 7
            obs_g = plsc.load_gather(obs_v, [tp_i, sp_i]).astype(jnp.float32)
            al_g = plsc.load_gather(alpha_v, [sp_i, sp_i])
            dk = plsc.load_gather(decay_v, [jnp.abs(t_i - tp_i)])
            plsc.store_scatter(table_v, [e * N_S + sp_i], al_g * obs_g * dk)

        def do_chunk(c, bank, sem):
            for cp in in_copies(c, bank, sem):
                cp.wait()
            pltpu.make_async_copy(out_hbm.at[pl.ds(wid * PER_W, CHUNK)],
                                  out_v, sem_out).wait()
            tp_v, sp_v, t_v, s_v = bank

            @plsc.parallel_loop(0, CHUNK // L, unroll=16)
            def _gather(k):
                sl = pl.ds(k * L, L)
                flat = ((tp_v[sl] * N_S + sp_v[sl]) * N_T + t_v[sl]) * N_S + s_v[sl]
                out_v[sl] = plsc.load_gather(table_v, [flat])

            base = wid * PER_W + c * CHUNK
            pltpu.make_async_copy(out_v, out_hbm.at[pl.ds(base, CHUNK)],
                                  sem_out).start()

        def pair(j, _):
            c0 = 2 * j
            for cp in in_copies(c0 + 1, banks[1], sem_b):
                cp.start()
            do_chunk(c0, banks[0], sem_a)

            @pl.when(c0 + 2 < N_CHUNKS)
            def _():
                for cp in in_copies(c0 + 2, banks[0], sem_a):
                    cp.start()
            do_chunk(c0 + 1, banks[1], sem_b)
            return 0

        lax.fori_loop(0, N_PAIRS, pair, 0)
        pltpu.make_async_copy(
            out_v, out_hbm.at[pl.ds(wid * PER_W + (N_CHUNKS - 1) * CHUNK, CHUNK)],
            sem_out).wait()

    return pl.kernel(
        sc_kernel,
        out_type=jax.ShapeDtypeStruct((B,), jnp.float32),
        mesh=mesh,
        compiler_params=pltpu.CompilerParams(needs_layout_passes=False),
        scratch_types=[
            pltpu.VMEM((TBL,), jnp.float32),
            pltpu.VMEM((N_T, N_S), jnp.int32),
            pltpu.VMEM((N_S, N_S), jnp.float32),
            pltpu.VMEM((1,), jnp.float32),
            pltpu.VMEM((L,), jnp.float32),
            chunk_i32, chunk_i32, chunk_i32, chunk_i32,
            chunk_i32, chunk_i32, chunk_i32, chunk_i32,
            pltpu.VMEM((CHUNK,), jnp.float32),
            pltpu.SemaphoreType.DMA,
            pltpu.SemaphoreType.DMA,
            pltpu.SemaphoreType.DMA,
            pltpu.SemaphoreType.DMA,
        ],
    )


_SC_CALL = _make_sc_call()


def kernel(tp, sp, t, s, obs, alpha, beta):
    return _SC_CALL(tp, sp, t, s, obs, alpha, beta)


# speedup vs baseline: 1.0287x; 1.0151x over previous
import jax
import jax.numpy as jnp
from jax import lax
from jax.experimental import pallas as pl
from jax.experimental.pallas import tpu as pltpu, tpu_sc as plsc

B = 1048576
N_T = 8
N_S = 8
TBL = N_T * N_S * N_T * N_S

NC = 2
NS = 16
L = 16
NW = NC * NS
PER_W = B // NW
CHUNK = 8192
N_CHUNKS = PER_W // CHUNK
N_PAIRS = N_CHUNKS // 2


def _make_sc_call():
    mesh = plsc.VectorSubcoreMesh(core_axis_name="c", subcore_axis_name="s")

    chunk_i32 = pltpu.VMEM((CHUNK,), jnp.int32)

    def sc_kernel(tp_hbm, sp_hbm, t_hbm, s_hbm, obs_hbm, alpha_hbm, beta_hbm,
                  out_hbm,
                  table_v, obs_v, alpha_v, beta_v, decay_v,
                  tp0, sp0, t0, s0,
                  tp1, sp1, t1, s1,
                  out_v,
                  sem_p, sem_a, sem_b, sem_out):
        wid = lax.axis_index("s") * NC + lax.axis_index("c")
        banks = ((tp0, sp0, t0, s0), (tp1, sp1, t1, s1))
        srcs = (tp_hbm, sp_hbm, t_hbm, s_hbm)

        def in_copies(c, bank, sem):
            base = wid * PER_W + c * CHUNK
            return [pltpu.make_async_copy(src.at[pl.ds(base, CHUNK)], dst, sem)
                    for src, dst in zip(srcs, bank)]

        c1 = pltpu.make_async_copy(obs_hbm, obs_v, sem_p)
        c2 = pltpu.make_async_copy(alpha_hbm, alpha_v, sem_p)
        c3 = pltpu.make_async_copy(beta_hbm, beta_v, sem_p)
        c1.start(); c2.start(); c3.start()
        for cp in in_copies(0, banks[0], sem_a):
            cp.start()
        c1.wait(); c2.wait(); c3.wait()

        lane = lax.iota(jnp.int32, L)
        beta = plsc.load_gather(beta_v, [lane & 0])
        decay_v[pl.ds(0, L)] = beta * jnp.exp(-beta * lane.astype(jnp.float32))

        zeros = jnp.zeros((L,), jnp.float32)

        @plsc.parallel_loop(0, TBL // L, unroll=8)
        def _zero(i):
            table_v[pl.ds(i * L, L)] = zeros

        @plsc.parallel_loop(0, N_T * N_S * N_T // L, unroll=4)
        def _build(i):
            e = i * L + lane
            tp_i = e >> 6
            sp_i = (e >> 3) & 7
            t_i = e & 7
            obs_g = plsc.load_gather(obs_v, [tp_i, sp_i]).astype(jnp.float32)
            al_g = plsc.load_gather(alpha_v, [sp_i, sp_i])
            dk = plsc.load_gather(decay_v, [jnp.abs(t_i - tp_i)])
            plsc.store_scatter(table_v, [e * N_S + sp_i], al_g * obs_g * dk)

        def do_chunk(c, bank, sem, out_wait_pred=None):
            for cp in in_copies(c, bank, sem):
                cp.wait()
            out_wait = pltpu.make_async_copy(
                out_hbm.at[pl.ds(wid * PER_W, CHUNK)], out_v, sem_out).wait
            if out_wait_pred is None:
                out_wait()
            else:
                pl.when(out_wait_pred)(out_wait)
            tp_v, sp_v, t_v, s_v = bank

            @plsc.parallel_loop(0, CHUNK // L, unroll=16)
            def _gather(k):
                sl = pl.ds(k * L, L)
                flat = ((tp_v[sl] * N_S + sp_v[sl]) * N_T + t_v[sl]) * N_S + s_v[sl]
                out_v[sl] = plsc.load_gather(table_v, [flat])

            base = wid * PER_W + c * CHUNK
            pltpu.make_async_copy(out_v, out_hbm.at[pl.ds(base, CHUNK)],
                                  sem_out).start()

        def pair(j, _):
            c0 = 2 * j
            for cp in in_copies(c0 + 1, banks[1], sem_b):
                cp.start()
            do_chunk(c0, banks[0], sem_a, out_wait_pred=j > 0)

            @pl.when(c0 + 2 < N_CHUNKS)
            def _():
                for cp in in_copies(c0 + 2, banks[0], sem_a):
                    cp.start()
            do_chunk(c0 + 1, banks[1], sem_b)
            return 0

        lax.fori_loop(0, N_PAIRS, pair, 0)
        pltpu.make_async_copy(
            out_v, out_hbm.at[pl.ds(wid * PER_W + (N_CHUNKS - 1) * CHUNK, CHUNK)],
            sem_out).wait()

    return pl.kernel(
        sc_kernel,
        out_type=jax.ShapeDtypeStruct((B,), jnp.float32),
        mesh=mesh,
        compiler_params=pltpu.CompilerParams(needs_layout_passes=False),
        scratch_types=[
            pltpu.VMEM((TBL,), jnp.float32),
            pltpu.VMEM((N_T, N_S), jnp.int32),
            pltpu.VMEM((N_S, N_S), jnp.float32),
            pltpu.VMEM((1,), jnp.float32),
            pltpu.VMEM((L,), jnp.float32),
            chunk_i32, chunk_i32, chunk_i32, chunk_i32,
            chunk_i32, chunk_i32, chunk_i32, chunk_i32,
            pltpu.VMEM((CHUNK,), jnp.float32),
            pltpu.SemaphoreType.DMA,
            pltpu.SemaphoreType.DMA,
            pltpu.SemaphoreType.DMA,
            pltpu.SemaphoreType.DMA,
        ],
    )


_SC_CALL = _make_sc_call()


def kernel(tp, sp, t, s, obs, alpha, beta):
    return _SC_CALL(tp, sp, t, s, obs, alpha, beta)


# speedup vs baseline: 1.0560x; 1.0265x over previous
import jax
import jax.numpy as jnp
from jax import lax
from jax.experimental import pallas as pl
from jax.experimental.pallas import tpu as pltpu, tpu_sc as plsc

B = 1048576
N_T = 8
N_S = 8
TBL = N_T * N_S * N_T * N_S

NC = 2
NS = 16
L = 16
NW = NC * NS
PER_W = B // NW
CHUNK = 8192
N_CHUNKS = PER_W // CHUNK
N_PAIRS = N_CHUNKS // 2


def _make_sc_call():
    mesh = plsc.VectorSubcoreMesh(core_axis_name="c", subcore_axis_name="s")

    chunk_i32 = pltpu.VMEM((CHUNK,), jnp.int32)

    def sc_kernel(tp_hbm, sp_hbm, t_hbm, s_hbm, obs_hbm, alpha_hbm, beta_hbm,
                  out_hbm,
                  table_v, obs_v, alpha_v, beta_v, decay_v,
                  tp0, sp0, t0, s0,
                  tp1, sp1, t1, s1,
                  out_a, out_b,
                  sem_p, sem_a, sem_b, sem_oa, sem_ob):
        wid = lax.axis_index("s") * NC + lax.axis_index("c")
        banks = ((tp0, sp0, t0, s0), (tp1, sp1, t1, s1))
        srcs = (tp_hbm, sp_hbm, t_hbm, s_hbm)

        def in_copies(c, bank, sem):
            base = wid * PER_W + c * CHUNK
            return [pltpu.make_async_copy(src.at[pl.ds(base, CHUNK)], dst, sem)
                    for src, dst in zip(srcs, bank)]

        c1 = pltpu.make_async_copy(obs_hbm, obs_v, sem_p)
        c2 = pltpu.make_async_copy(alpha_hbm, alpha_v, sem_p)
        c3 = pltpu.make_async_copy(beta_hbm, beta_v, sem_p)
        c1.start(); c2.start(); c3.start()
        for cp in in_copies(0, banks[0], sem_a):
            cp.start()
        c1.wait(); c2.wait(); c3.wait()

        lane = lax.iota(jnp.int32, L)
        beta = plsc.load_gather(beta_v, [lane & 0])
        decay_v[pl.ds(0, L)] = beta * jnp.exp(-beta * lane.astype(jnp.float32))

        zeros = jnp.zeros((L,), jnp.float32)

        @plsc.parallel_loop(0, TBL // L, unroll=8)
        def _zero(i):
            table_v[pl.ds(i * L, L)] = zeros

        @plsc.parallel_loop(0, N_T * N_S * N_T // L, unroll=4)
        def _build(i):
            e = i * L + lane
            tp_i = e >> 6
            sp_i = (e >> 3) & 7
            t_i = e & 7
            obs_g = plsc.load_gather(obs_v, [tp_i, sp_i]).astype(jnp.float32)
            al_g = plsc.load_gather(alpha_v, [sp_i, sp_i])
            dk = plsc.load_gather(decay_v, [jnp.abs(t_i - tp_i)])
            plsc.store_scatter(table_v, [e * N_S + sp_i], al_g * obs_g * dk)

        def do_chunk(c, bank, sem, out_v, sem_out, out_wait_pred):
            for cp in in_copies(c, bank, sem):
                cp.wait()
            out_wait = pltpu.make_async_copy(
                out_hbm.at[pl.ds(wid * PER_W, CHUNK)], out_v, sem_out).wait
            pl.when(out_wait_pred)(out_wait)
            tp_v, sp_v, t_v, s_v = bank

            @plsc.parallel_loop(0, CHUNK // L, unroll=16)
            def _gather(k):
                sl = pl.ds(k * L, L)
                flat = ((tp_v[sl] * N_S + sp_v[sl]) * N_T + t_v[sl]) * N_S + s_v[sl]
                out_v[sl] = plsc.load_gather(table_v, [flat])

            base = wid * PER_W + c * CHUNK
            pltpu.make_async_copy(out_v, out_hbm.at[pl.ds(base, CHUNK)],
                                  sem_out).start()

        def pair(j, _):
            c0 = 2 * j
            for cp in in_copies(c0 + 1, banks[1], sem_b):
                cp.start()
            do_chunk(c0, banks[0], sem_a, out_a, sem_oa, j > 0)

            @pl.when(c0 + 2 < N_CHUNKS)
            def _():
                for cp in in_copies(c0 + 2, banks[0], sem_a):
                    cp.start()
            do_chunk(c0 + 1, banks[1], sem_b, out_b, sem_ob, j > 0)
            return 0

        lax.fori_loop(0, N_PAIRS, pair, 0)
        pltpu.make_async_copy(
            out_a, out_hbm.at[pl.ds(wid * PER_W + (N_CHUNKS - 2) * CHUNK, CHUNK)],
            sem_oa).wait()
        pltpu.make_async_copy(
            out_b, out_hbm.at[pl.ds(wid * PER_W + (N_CHUNKS - 1) * CHUNK, CHUNK)],
            sem_ob).wait()

    return pl.kernel(
        sc_kernel,
        out_type=jax.ShapeDtypeStruct((B,), jnp.float32),
        mesh=mesh,
        compiler_params=pltpu.CompilerParams(needs_layout_passes=False),
        scratch_types=[
            pltpu.VMEM((TBL,), jnp.float32),
            pltpu.VMEM((N_T, N_S), jnp.int32),
            pltpu.VMEM((N_S, N_S), jnp.float32),
            pltpu.VMEM((1,), jnp.float32),
            pltpu.VMEM((L,), jnp.float32),
            chunk_i32, chunk_i32, chunk_i32, chunk_i32,
            chunk_i32, chunk_i32, chunk_i32, chunk_i32,
            pltpu.VMEM((CHUNK,), jnp.float32),
            pltpu.VMEM((CHUNK,), jnp.float32),
            pltpu.SemaphoreType.DMA,
            pltpu.SemaphoreType.DMA,
            pltpu.SemaphoreType.DMA,
            pltpu.SemaphoreType.DMA,
            pltpu.SemaphoreType.DMA,
        ],
    )


_SC_CALL = _make_sc_call()


def kernel(tp, sp, t, s, obs, alpha, beta):
    return _SC_CALL(tp, sp, t, s, obs, alpha, beta)
